# Initial kernel scaffold; baseline (speedup 1.0000x reference)
import jax
import jax.numpy as jnp
from jax.experimental import pallas as pl


def kernel(node_ids, messages, timestamps, n_nodes):
    raise NotImplementedError("write your pallas kernel here")


# speedup vs baseline: 1.7407x; 1.7407x over previous
import functools

import jax
import jax.numpy as jnp
from jax import lax
from jax.experimental import pallas as pl
from jax.experimental.pallas import tpu as pltpu
from jax.experimental.pallas import tpu_sc as plsc

NE = 200000
ND = 128
NNODES = 100000
NW = 32
NT = 3136
NPAD = NW * NT
DUMP = NPAD
ECH = 8000
NCH = NE // ECH
GROUPS = NT // 16
TBUF = 3200
NTR = TBUF // 128


def _shift_up(x, lane):
    idx = jnp.minimum(lane + 1, 15).reshape(16, 1)
    return lax.gather(
        x, idx,
        dimension_numbers=lax.GatherDimensionNumbers(
            offset_dims=(), collapsed_slice_dims=(0,), start_index_map=(0,)),
        slice_sizes=(1,),
        mode=lax.GatherScatterMode.PROMISE_IN_BOUNDS)


def _body(nid_hbm, msg_hbm, ts_hbm, zeros_hbm, nn_hbm,
          msg_out, ts_out, vi_out,
          lp_ref, ev_ref, nn_ref, ts_idx, vi_buf, msrc, mdst, idst,
          mdst2, idst2, ts_buf, rows, zbuf, sem):
    wid = lax.axis_index("s") * 2 + lax.axis_index("c")
    lo = wid * NT
    lane = lax.iota(jnp.int32, 16)

    pltpu.sync_copy(nn_hbm, nn_ref)
    pltpu.sync_copy(zeros_hbm, zbuf)

    neg1 = jnp.full((16,), -1, jnp.int32)
    zero16 = jnp.zeros((16,), jnp.int32)
    dump16 = jnp.full((16,), DUMP, jnp.int32)

    def init_lp(g, _):
        lp_ref[pl.ds(g * 16, 16)] = neg1
        return 0
    lax.fori_loop(0, GROUPS, init_lp, 0)

    def init_bufs(g, _):
        msrc[pl.ds(g * 16, 16)] = zero16
        ts_idx[pl.ds(g * 16, 16)] = zero16
        mdst[pl.ds(g * 16, 16)] = dump16
        idst[pl.ds(g * 16, 16)] = dump16
        return 0
    lax.fori_loop(0, TBUF // 16, init_bufs, 0)

    def ev_chunk(c, _):
        pltpu.sync_copy(nid_hbm.at[pl.ds(c * ECH, ECH)], ev_ref)
        base = c * ECH

        def ev_vec(i, _):
            nid = ev_ref[pl.ds(i * 16, 16)]
            pos = base + i * 16 + lane
            key = lax.shift_left(nid, 4) + lane
            skey, spos = lax.sort([key, pos], num_keys=1)
            snid = lax.shift_right_arithmetic(skey, 4)
            nxt = _shift_up(snid, lane)
            is_last = (snid != nxt) | (lane == 15)
            local = snid - lo
            m = is_last & (local >= 0) & (local < NT)
            localc = jnp.clip(local, 0, NT - 1)
            plsc.store_scatter(lp_ref, [localc], spos, mask=m)
            return 0
        lax.fori_loop(0, ECH // 16, ev_vec, 0)
        return 0
    lax.fori_loop(0, NCH, ev_chunk, 0)

    nn = nn_ref[pl.ds(0, 16)]

    def a_body(g, carry):
        nv, ni = carry
        lp = lp_ref[pl.ds(g * 16, 16)]
        node = lo + g * 16 + lane
        valid = (lp >= 0) & (node < nn)
        safe = jnp.maximum(lp, 0)
        ts_idx[pl.ds(g * 16, 16)] = safe
        vi_buf[pl.ds(g * 16, 16)] = jnp.where(valid, 1, 0)
        plsc.store_compressed(msrc.at[pl.ds(nv, 16)], safe, mask=valid)
        plsc.store_compressed(mdst.at[pl.ds(nv, 16)], node, mask=valid)
        inv = ~valid
        plsc.store_compressed(idst.at[pl.ds(ni, 16)], node, mask=inv)
        cv = jnp.max(plsc.all_reduce_population_count(valid))
        return nv + cv, ni + (16 - cv)

    nv, ni = lax.fori_loop(0, GROUPS, a_body, (jnp.int32(0), jnp.int32(0)))

    def c_body(j, _):
        for q in range(8):
            mdst2[j, pl.ds(q * 16, 16)] = mdst[pl.ds(j * 128 + q * 16, 16)]
            idst2[j, pl.ds(q * 16, 16)] = idst[pl.ds(j * 128 + q * 16, 16)]
        return 0
    lax.fori_loop(0, NTR, c_body, 0)

    def ts_g(j, _):
        pltpu.async_copy(ts_hbm.at[ts_idx.at[pl.ds(j * 128, 128)]],
                         ts_buf.at[pl.ds(j * 128, 128)], sem).wait()
        return 0
    lax.fori_loop(0, NTR, ts_g, 0)

    def ts_m(g, _):
        v = vi_buf[pl.ds(g * 16, 16)].astype(jnp.float32)
        ts_buf[pl.ds(g * 16, 16)] = ts_buf[pl.ds(g * 16, 16)] * v
        return 0
    lax.fori_loop(0, GROUPS, ts_m, 0)
    pltpu.sync_copy(ts_buf.at[pl.ds(0, NT)], ts_out.at[pl.ds(lo, NT)])
    pltpu.sync_copy(vi_buf.at[pl.ds(0, NT)], vi_out.at[pl.ds(lo, NT)])

    nvt = (nv + 127) // 128

    def m_body(j, _):
        pltpu.async_copy(msg_hbm.at[msrc.at[pl.ds(j * 128, 128)]],
                         rows, sem).wait()
        pltpu.async_copy(rows, msg_out.at[mdst2.at[j]], sem).wait()
        return 0
    lax.fori_loop(0, nvt, m_body, 0)

    nit = (ni + 127) // 128

    def z_body(j, _):
        pltpu.async_copy(zbuf, msg_out.at[idst2.at[j]], sem).wait()
        return 0
    lax.fori_loop(0, nit, z_body, 0)


_sc_call = pl.kernel(
    _body,
    out_type=[
        jax.ShapeDtypeStruct((NPAD + 1, ND), jnp.float32),
        jax.ShapeDtypeStruct((NPAD,), jnp.float32),
        jax.ShapeDtypeStruct((NPAD,), jnp.int32),
    ],
    mesh=plsc.VectorSubcoreMesh(core_axis_name="c", subcore_axis_name="s"),
    compiler_params=pltpu.CompilerParams(needs_layout_passes=False),
    scratch_types=[
        pltpu.VMEM((NT,), jnp.int32),
        pltpu.VMEM((ECH,), jnp.int32),
        pltpu.VMEM((16,), jnp.int32),
        pltpu.VMEM((TBUF,), jnp.int32),
        pltpu.VMEM((TBUF,), jnp.int32),
        pltpu.VMEM((TBUF,), jnp.int32),
        pltpu.VMEM((TBUF,), jnp.int32),
        pltpu.VMEM((TBUF,), jnp.int32),
        pltpu.VMEM((NTR, 128), jnp.int32),
        pltpu.VMEM((NTR, 128), jnp.int32),
        pltpu.VMEM((TBUF,), jnp.float32),
        pltpu.VMEM((128, ND), jnp.float32),
        pltpu.VMEM((128, ND), jnp.float32),
        pltpu.SemaphoreType.DMA,
    ],
)


def kernel(node_ids, messages, timestamps, n_nodes):
    zeros = jnp.zeros((128, ND), jnp.float32)
    nn = jnp.full((16,), n_nodes, jnp.int32)
    msg_o, ts_o, vi_o = _sc_call(node_ids, messages, timestamps, zeros, nn)
    return msg_o[:NNODES], ts_o[:NNODES], vi_o[:NNODES] > 0


# speedup vs baseline: 2.5740x; 1.4787x over previous
import jax
import jax.numpy as jnp
from jax import lax
from jax.experimental import pallas as pl
from jax.experimental.pallas import tpu as pltpu
from jax.experimental.pallas import tpu_sc as plsc

NE = 200000
ND = 128
NNODES = 100000
NW = 32
NT = 3136
NPAD = NW * NT
DUMP = NPAD
GROUPS = NT // 16
TBUF = 3328
NTR = TBUF // 128
ECH1 = 6256
NEPAD = NW * ECH1
EV1 = ECH1 // 16


def _shift_up(x, lane):
    idx = jnp.minimum(lane + 1, 15).reshape(16, 1)
    return lax.gather(
        x, idx,
        dimension_numbers=lax.GatherDimensionNumbers(
            offset_dims=(), collapsed_slice_dims=(0,), start_index_map=(0,)),
        slice_sizes=(1,),
        mode=lax.GatherScatterMode.PROMISE_IN_BOUNDS)


def _body1(nid_hbm, partials_out, lp_ref, ev_ref):
    wid = lax.axis_index("s") * 2 + lax.axis_index("c")
    lane = lax.iota(jnp.int32, 16)
    neg1 = jnp.full((16,), -1, jnp.int32)

    def init_lp(g, _):
        for q in range(4):
            lp_ref[pl.ds(g * 64 + q * 16, 16)] = neg1
        return 0
    lax.fori_loop(0, NPAD // 64, init_lp, 0)

    base = wid * ECH1
    pltpu.sync_copy(nid_hbm.at[pl.ds(base, ECH1)], ev_ref)

    def ev_vec(i, _):
        nid = ev_ref[pl.ds(i * 16, 16)]
        pos = base + i * 16 + lane
        key = lax.shift_left(nid, 4) + lane
        skey, spos = lax.sort([key, pos], num_keys=1)
        snid = lax.shift_right_arithmetic(skey, 4)
        nxt = _shift_up(snid, lane)
        is_last = (snid != nxt) | (lane == 15)
        m = is_last & (spos < NE)
        localc = jnp.clip(snid, 0, NPAD - 1)
        plsc.store_scatter(lp_ref, [localc], spos, mask=m)
        return 0
    lax.fori_loop(0, EV1, ev_vec, 0)

    pltpu.sync_copy(lp_ref, partials_out.at[pl.ds(wid * NPAD, NPAD)])


_k1 = pl.kernel(
    _body1,
    out_type=[jax.ShapeDtypeStruct((NW * NPAD,), jnp.int32)],
    mesh=plsc.VectorSubcoreMesh(core_axis_name="c", subcore_axis_name="s"),
    compiler_params=pltpu.CompilerParams(needs_layout_passes=False),
    scratch_types=[
        pltpu.VMEM((NPAD,), jnp.int32),
        pltpu.VMEM((ECH1,), jnp.int32),
    ],
)


def _body2(partials, msg_hbm, ts_hbm, zeros_hbm, nn_hbm,
           msg_out, ts_out, vi_out,
           lp_ref, pb0, pb1, pb2, pb3, pb4, pb5, pb6, pb7,
           nn_ref, ts_idx, vi_buf, msrc, mdst, idst,
           mdst2, idst2, ts_buf, rows_a, rows_b, zbuf, semg, sems, semt):
    pbufs = [pb0, pb1, pb2, pb3, pb4, pb5, pb6, pb7]
    wid = lax.axis_index("s") * 2 + lax.axis_index("c")
    lo = wid * NT
    lane = lax.iota(jnp.int32, 16)

    pltpu.sync_copy(nn_hbm, nn_ref)
    pltpu.sync_copy(zeros_hbm, zbuf)

    neg1 = jnp.full((16,), -1, jnp.int32)
    zero16 = jnp.zeros((16,), jnp.int32)
    dump16 = jnp.full((16,), DUMP, jnp.int32)

    def init_lp(g, _):
        lp_ref[pl.ds(g * 16, 16)] = neg1
        return 0
    lax.fori_loop(0, GROUPS, init_lp, 0)

    def init_bufs(g, _):
        msrc[pl.ds(g * 16, 16)] = zero16
        ts_idx[pl.ds(g * 16, 16)] = zero16
        mdst[pl.ds(g * 16, 16)] = dump16
        idst[pl.ds(g * 16, 16)] = dump16
        return 0
    lax.fori_loop(0, TBUF // 16, init_bufs, 0)

    for b in range(4):
        das = [pltpu.async_copy(
                   partials.at[pl.ds((b * 8 + r) * NPAD + lo, NT)],
                   pbufs[r], semt)
               for r in range(8)]
        for d in das:
            d.wait()

        def mg(g, _):
            acc = lp_ref[pl.ds(g * 16, 16)]
            for r in range(8):
                acc = jnp.maximum(acc, pbufs[r][pl.ds(g * 16, 16)])
            lp_ref[pl.ds(g * 16, 16)] = acc
            return 0
        lax.fori_loop(0, GROUPS, mg, 0)

    nn = nn_ref[pl.ds(0, 16)]

    def a_body(g, carry):
        nv, ni = carry
        lp = lp_ref[pl.ds(g * 16, 16)]
        node = lo + g * 16 + lane
        valid = (lp >= 0) & (node < nn)
        safe = jnp.maximum(lp, 0)
        ts_idx[pl.ds(g * 16, 16)] = safe
        vi_buf[pl.ds(g * 16, 16)] = jnp.where(valid, 1, 0)
        plsc.store_compressed(msrc.at[pl.ds(nv, 16)], safe, mask=valid)
        plsc.store_compressed(mdst.at[pl.ds(nv, 16)], node, mask=valid)
        inv = ~valid
        plsc.store_compressed(idst.at[pl.ds(ni, 16)], node, mask=inv)
        cv = jnp.max(plsc.all_reduce_population_count(valid))
        return nv + cv, ni + (16 - cv)

    nv, ni = lax.fori_loop(0, GROUPS, a_body, (jnp.int32(0), jnp.int32(0)))

    def c_body(j, _):
        for q in range(8):
            mdst2[j, pl.ds(q * 16, 16)] = mdst[pl.ds(j * 128 + q * 16, 16)]
            idst2[j, pl.ds(q * 16, 16)] = idst[pl.ds(j * 128 + q * 16, 16)]
        return 0
    lax.fori_loop(0, NTR, c_body, 0)

    def ts_f(j, _):
        pltpu.async_copy(ts_hbm.at[ts_idx.at[pl.ds(j * 128, 128)]],
                         ts_buf.at[pl.ds(j * 128, 128)], semt)
        return 0
    lax.fori_loop(0, 25, ts_f, 0)

    nvt = (nv + 127) // 128

    def m_body(j, _):
        ja, jb = 2 * j, 2 * j + 1
        da = pltpu.async_copy(
            msg_hbm.at[msrc.at[pl.ds(ja * 128, 128)]], rows_a, semg)
        db = pltpu.async_copy(
            msg_hbm.at[msrc.at[pl.ds(jb * 128, 128)]], rows_b, semg)
        da.wait()
        sa = pltpu.async_copy(rows_a, msg_out.at[mdst2.at[ja]], sems)
        db.wait()
        sa.wait()
        sb = pltpu.async_copy(rows_b, msg_out.at[mdst2.at[jb]], sems)
        sb.wait()
        return 0
    lax.fori_loop(0, (nvt + 1) // 2, m_body, 0)

    nit = (ni + 127) // 128

    def z_f(j, _):
        pltpu.async_copy(zbuf, msg_out.at[idst2.at[j]], sems)
        return 0
    lax.fori_loop(0, nit, z_f, 0)

    def ts_d(j, _):
        pltpu.make_async_copy(ts_hbm.at[pl.ds(0, 128)],
                              ts_buf.at[pl.ds(j * 128, 128)], semt).wait()
        return 0
    lax.fori_loop(0, 25, ts_d, 0)

    def ts_m(g, _):
        v = vi_buf[pl.ds(g * 16, 16)].astype(jnp.float32)
        ts_buf[pl.ds(g * 16, 16)] = ts_buf[pl.ds(g * 16, 16)] * v
        return 0
    lax.fori_loop(0, GROUPS, ts_m, 0)
    pltpu.sync_copy(ts_buf.at[pl.ds(0, NT)], ts_out.at[pl.ds(lo, NT)])
    pltpu.sync_copy(vi_buf.at[pl.ds(0, NT)], vi_out.at[pl.ds(lo, NT)])

    def z_d(j, _):
        pltpu.make_async_copy(zbuf, msg_out.at[idst2.at[j]], sems).wait()
        return 0
    lax.fori_loop(0, nit, z_d, 0)


_k2 = pl.kernel(
    _body2,
    out_type=[
        jax.ShapeDtypeStruct((NPAD + 1, ND), jnp.float32),
        jax.ShapeDtypeStruct((NPAD,), jnp.float32),
        jax.ShapeDtypeStruct((NPAD,), jnp.int32),
    ],
    mesh=plsc.VectorSubcoreMesh(core_axis_name="c", subcore_axis_name="s"),
    compiler_params=pltpu.CompilerParams(needs_layout_passes=False),
    scratch_types=[
        pltpu.VMEM((NT,), jnp.int32),
        pltpu.VMEM((NT,), jnp.int32),
        pltpu.VMEM((NT,), jnp.int32),
        pltpu.VMEM((NT,), jnp.int32),
        pltpu.VMEM((NT,), jnp.int32),
        pltpu.VMEM((NT,), jnp.int32),
        pltpu.VMEM((NT,), jnp.int32),
        pltpu.VMEM((NT,), jnp.int32),
        pltpu.VMEM((NT,), jnp.int32),
        pltpu.VMEM((16,), jnp.int32),
        pltpu.VMEM((TBUF,), jnp.int32),
        pltpu.VMEM((TBUF,), jnp.int32),
        pltpu.VMEM((TBUF,), jnp.int32),
        pltpu.VMEM((TBUF,), jnp.int32),
        pltpu.VMEM((TBUF,), jnp.int32),
        pltpu.VMEM((NTR, 128), jnp.int32),
        pltpu.VMEM((NTR, 128), jnp.int32),
        pltpu.VMEM((TBUF,), jnp.float32),
        pltpu.VMEM((128, ND), jnp.float32),
        pltpu.VMEM((128, ND), jnp.float32),
        pltpu.VMEM((128, ND), jnp.float32),
        pltpu.SemaphoreType.DMA,
        pltpu.SemaphoreType.DMA,
        pltpu.SemaphoreType.DMA,
    ],
)


def kernel(node_ids, messages, timestamps, n_nodes):
    nid_pad = jnp.concatenate(
        [node_ids, jnp.zeros((NEPAD - NE,), jnp.int32)])
    (partials,) = _k1(nid_pad)
    zeros = jnp.zeros((128, ND), jnp.float32)
    nn = jnp.full((16,), n_nodes, jnp.int32)
    msg_o, ts_o, vi_o = _k2(partials, messages, timestamps, zeros, nn)
    return msg_o[:NNODES], ts_o[:NNODES], vi_o[:NNODES] > 0
